# baseline (speedup 1.0000x reference)
import functools

import jax
import jax.numpy as jnp
from jax import lax
from jax.experimental import pallas as pl
from jax.experimental.pallas import tpu as pltpu
from jax.experimental.pallas import tpu_sc as plsc

N = 10000
D = 128
E = 160000
L = 4
OUT = 128

NP = 10240
EP = 163840
NC = 2
NS = 16
NW = NC * NS
PER_TILE = EP // NW
K = 128
CHUNKS = PER_TILE // K
CH0 = 80
CH1 = 80 - CH0
CHMAX = max(CH0, CH1)
NCHROW = EP // K
RPT = NP // NS
P8 = 8

_mesh = plsc.VectorSubcoreMesh(core_axis_name="c", subcore_axis_name="s")


def _f32(*shape):
    return jax.ShapeDtypeStruct(shape, jnp.float32)


def _silu(x):
    return x * jax.nn.sigmoid(x)


@functools.partial(
    pl.kernel,
    out_type=(_f32(EP, D), _f32(EP, P8), _f32(EP, P8)),
    mesh=_mesh,
    compiler_params=pltpu.CompilerParams(use_tc_tiling_on_sc=False),
    scratch_types=(
        pltpu.VMEM((CHMAX, K), jnp.int32),
        pltpu.VMEM((CHMAX, K), jnp.int32),
        pltpu.VMEM((K, D), jnp.float32),
        pltpu.VMEM((K, D), jnp.float32),
        pltpu.VMEM((K, D), jnp.float32),
        pltpu.VMEM((K, D), jnp.float32),
        pltpu.VMEM((K, P8), jnp.float32),
        pltpu.VMEM((K, P8), jnp.float32),
        pltpu.VMEM((K, P8), jnp.float32),
        pltpu.VMEM((K, P8), jnp.float32),
        pltpu.SemaphoreType.DMA,
        pltpu.SemaphoreType.DMA,
        pltpu.SemaphoreType.DMA,
        pltpu.SemaphoreType.DMA,
        pltpu.SemaphoreType.DMA,
    ),
)
def _sc_gather(srcp2, dstp2, a_t, b_t, ps_t, pd_t, s_out, ps_out, pd_out,
               sidxs, didxs, ab0, ab1, bb0, bb1, ps0, ps1, pd0, pd1,
               gs0, gs1, os0, os1, isem):
    cid = lax.axis_index("c")
    sid = lax.axis_index("s")
    ch_mine = jnp.where(cid == 0, CH0, CH1)
    crow = sid * ch_mine + cid * (NS * CH0)
    base = crow * K
    crow_ld = jnp.minimum(crow, NCHROW - CHMAX)

    ci1 = pltpu.async_copy(srcp2.at[pl.ds(crow_ld, CHMAX)], sidxs, isem)
    ci2 = pltpu.async_copy(dstp2.at[pl.ds(crow_ld, CHMAX)], didxs, isem)
    ci1.wait()
    ci2.wait()

    bufs = ((ab0, bb0, ps0, pd0, gs0, os0), (ab1, bb1, ps1, pd1, gs1, os1))

    def fire(ci, b):
        ab, bb, ps, pd, gs, _ = bufs[b]
        return (pltpu.async_copy(a_t.at[didxs.at[ci]], ab, gs),
                pltpu.async_copy(b_t.at[sidxs.at[ci]], bb, gs),
                pltpu.async_copy(ps_t.at[sidxs.at[ci]], ps, gs),
                pltpu.async_copy(pd_t.at[didxs.at[ci]], pd, gs))

    def consume(ci, b, gds):
        ab, bb, ps, pd, _, osm = bufs[b]
        for g in gds:
            g.wait()

        @pl.loop(0, K, unroll=4)
        def _row(e):
            for c in range(D // 16):
                sl = pl.ds(c * 16, 16)
                ab[e, sl] = ab[e, sl] + bb[e, sl]

        off = base + ci * K
        return (pltpu.async_copy(ab, s_out.at[pl.ds(off, K)], osm),
                pltpu.async_copy(ps, ps_out.at[pl.ds(off, K)], osm),
                pltpu.async_copy(pd, pd_out.at[pl.ds(off, K)], osm))

    pairs = ch_mine // 2

    @pl.loop(0, CHMAX // 2)
    def _pair(j):
        @pl.when(j < pairs)
        def _():
            c0 = 2 * j
            c1 = c0 + 1
            g0 = fire(c0, 0)
            g1 = fire(c1, 1)
            o0 = consume(c0, 0, g0)
            o1 = consume(c1, 1, g1)
            for o in o0 + o1:
                o.wait()


@functools.partial(
    pl.kernel,
    out_type=(_f32(NC, NP, D), _f32(NC, NP, P8)),
    mesh=_mesh,
    compiler_params=pltpu.CompilerParams(use_tc_tiling_on_sc=False),
    scratch_types=(
        pltpu.VMEM((CHUNKS, K), jnp.int32),
        pltpu.VMEM((K, D), jnp.float32),
        pltpu.VMEM((K, D), jnp.float32),
        pltpu.VMEM((K, P8), jnp.float32),
        pltpu.VMEM((K, P8), jnp.float32),
        pltpu.VMEM_SHARED((NP, D), jnp.float32),
        pltpu.VMEM_SHARED((NP, P8), jnp.float32),
        pltpu.SemaphoreType.DMA,
        pltpu.SemaphoreType.DMA,
        pltpu.SemaphoreType.DMA,
    ),
)
def _sc_scatter(m2, cd, dstp2, zd, zp, agg_out, dpos_out,
                didxs, mb0, mb1, cb0, cb1, agg_sh, dpos_sh, ls0, ls1, isem):
    cid = lax.axis_index("c")
    sid = lax.axis_index("s")
    wid = sid * NC + cid
    base = wid * PER_TILE
    crow = wid * CHUNKS
    row0 = sid * RPT

    ci = pltpu.async_copy(dstp2.at[pl.ds(crow, CHUNKS)], didxs, isem)

    pltpu.sync_copy(zd, mb0)
    pltpu.sync_copy(zp, cb0)
    for j in range(RPT // K):
        pltpu.sync_copy(mb0, agg_sh.at[pl.ds(row0 + j * K, K)])
        pltpu.sync_copy(cb0, dpos_sh.at[pl.ds(row0 + j * K, K)])
    ci.wait()
    plsc.subcore_barrier()

    bufs = ((mb0, cb0, ls0), (mb1, cb1, ls1))

    def fire(ci_, b):
        mb, cb, ls = bufs[b]
        off = base + ci_ * K
        return (pltpu.async_copy(m2.at[pl.ds(off, K)], mb, ls),
                pltpu.async_copy(cd.at[pl.ds(off, K)], cb, ls))

    def consume(ci_, b, lds):
        mb, cb, _ = bufs[b]
        for l_ in lds:
            l_.wait()
        pltpu.sync_copy(mb, agg_sh.at[didxs.at[ci_]], add=True)
        pltpu.sync_copy(cb, dpos_sh.at[didxs.at[ci_]], add=True)

    @pl.loop(0, CHUNKS // 2)
    def _pair(j):
        c0 = 2 * j
        c1 = c0 + 1
        l0 = fire(c0, 0)
        l1 = fire(c1, 1)
        consume(c0, 0, l0)
        consume(c1, 1, l1)

    plsc.subcore_barrier()
    for j in range(RPT // K):
        r = row0 + j * K
        pltpu.sync_copy(agg_sh.at[pl.ds(r, K)], mb0)
        pltpu.sync_copy(mb0, agg_out.at[cid, pl.ds(r, K)])
        pltpu.sync_copy(dpos_sh.at[pl.ds(r, K)], cb0)
        pltpu.sync_copy(cb0, dpos_out.at[cid, pl.ds(r, K)])


BN = 1024
GN = NP // BN
BE = 2048
GE = EP // BE


def _wspec(r, c):
    return pl.BlockSpec((r, c), lambda i: (0, 0))


def _nblk(c):
    return pl.BlockSpec((BN, c), lambda i: (i, 0))


def _eblk(c):
    return pl.BlockSpec((BE, c), lambda i: (i, 0))


def _dot(x, w):
    return jnp.dot(x, w, preferred_element_type=jnp.float32)


def _bf(x):
    return x.astype(jnp.bfloat16).astype(jnp.float32)


def _tc_embed(hp, w, b):
    def body(h_ref, w_ref, b_ref, o_ref):
        i = pl.program_id(0)
        x = _dot(h_ref[...], w_ref[...]) + b_ref[...]
        rid = i * BN + lax.broadcasted_iota(jnp.int32, (BN, 1), 0)
        o_ref[...] = jnp.where(rid < N, x, 0.0)

    return pl.pallas_call(
        body, grid=(GN,),
        in_specs=[_nblk(D), _wspec(D, D), _wspec(1, D)],
        out_specs=_nblk(D), out_shape=_f32(NP, D),
    )(hp, w, b)


def _tc_prep(hp, p8, w1a, w1b, wd, b1, w2, b2, cw1, cb1, cw2, cb2):
    def body(h_ref, p8_ref, w1a_r, w1b_r, wd_r, b1_r, w2_r, b2_r,
             cw1_r, cb1_r, cw2_r, cb2_r, a_o, b_o, ms_o, pc_o):
        x = h_ref[...]
        a = _dot(x, w1a_r[...])
        b = _dot(x, w1b_r[...])
        a_o[...] = a
        b_o[...] = b
        pre = _silu(a + b + _bf(1e-10 * jnp.ones((1, 1), jnp.float32)) * _bf(wd_r[...]) + b1_r[...])
        ms_o[...] = _silu(_dot(pre, w2_r[...]) + b2_r[...])
        cvec = _silu(_dot(x, cw1_r[...]) + cb1_r[...])
        cval = _dot(cvec, cw2_r[...]) + cb2_r[...]
        lane = lax.broadcasted_iota(jnp.int32, (BN, P8), 1)
        pc_o[...] = jnp.where(lane == 3, cval, p8_ref[...])

    return pl.pallas_call(
        body, grid=(GN,),
        in_specs=[_nblk(D), _nblk(P8), _wspec(D, D), _wspec(D, D),
                  _wspec(1, D), _wspec(1, D), _wspec(D, D), _wspec(1, D),
                  _wspec(D, D), _wspec(1, D), _wspec(D, 1), _wspec(1, 1)],
        out_specs=[_nblk(D), _nblk(D), _nblk(D), _nblk(P8)],
        out_shape=[_f32(NP, D), _f32(NP, D), _f32(NP, D), _f32(NP, P8)],
    )(hp, p8, w1a, w1b, wd, b1, w2, b2, cw1, cb1, cw2, cb2)


def _tc_edge(s, ps, pd, wd, b1, w2, b2):
    def body(s_ref, ps_ref, pd_ref, wd_r, b1_r, w2_r, b2_r, m2_o, cd_o):
        ps_v = ps_ref[...]
        pd_v = pd_ref[...]
        lane = lax.broadcasted_iota(jnp.int32, (BE, P8), 1)
        rel = jnp.where(lane < 3, ps_v - pd_v, 0.0)
        d2 = jnp.sum(rel * rel, axis=1, keepdims=True) + 1e-20
        dist = jnp.sqrt(d2)
        m1 = _silu(s_ref[...] + _bf(dist) * _bf(wd_r[...]) + b1_r[...])
        m2_o[...] = _silu(_dot(m1, w2_r[...]) + b2_r[...])
        cw = jnp.sum(jnp.where(lane == 3, ps_v, 0.0), axis=1, keepdims=True)
        cd_o[...] = cw * rel / (dist + 1e-8)

    return pl.pallas_call(
        body, grid=(GE,),
        in_specs=[_eblk(D), _eblk(P8), _eblk(P8),
                  _wspec(1, D), _wspec(1, D), _wspec(D, D), _wspec(1, D)],
        out_specs=[_eblk(D), _eblk(P8)],
        out_shape=[_f32(EP, D), _f32(EP, P8)],
    )(s, ps, pd, wd, b1, w2, b2)


def _tc_node(hp, aggp, mself, dposp, p8, nw1a, nw1b, nb1, nw2, nb2):
    def body(h_ref, agg_ref, ms_ref, dp_ref, p8_ref,
             nw1a_r, nw1b_r, nb1_r, nw2_r, nb2_r, h_o, p_o):
        i = pl.program_id(0)
        h = h_ref[...]
        agg = agg_ref[0] + agg_ref[1] + ms_ref[...]
        t = _silu(_dot(h, nw1a_r[...]) + _dot(agg, nw1b_r[...]) + nb1_r[...])
        upd = _dot(t, nw2_r[...]) + nb2_r[...]
        rid = i * BN + lax.broadcasted_iota(jnp.int32, (BN, 1), 0)
        mask = rid < N
        h_o[...] = h + jnp.where(mask, upd, 0.0)
        dp = dp_ref[0] + dp_ref[1]
        p_o[...] = p8_ref[...] + jnp.where(mask, dp, 0.0)

    return pl.pallas_call(
        body, grid=(GN,),
        in_specs=[_nblk(D),
                  pl.BlockSpec((NC, BN, D), lambda i: (0, i, 0)),
                  _nblk(D),
                  pl.BlockSpec((NC, BN, P8), lambda i: (0, i, 0)),
                  _nblk(P8),
                  _wspec(D, D), _wspec(D, D), _wspec(1, D),
                  _wspec(D, D), _wspec(1, D)],
        out_specs=[_nblk(D), _nblk(P8)],
        out_shape=[_f32(NP, D), _f32(NP, P8)],
    )(hp, aggp, mself, dposp, p8, nw1a, nw1b, nb1, nw2, nb2)


def _tc_head(hp, w1, b1, w2, b2):
    def body(h_ref, w1_r, b1_r, w2_r, b2_r, o_ref):
        t = _silu(_dot(h_ref[...], w1_r[...]) + b1_r[...])
        o_ref[...] = _dot(t, w2_r[...]) + b2_r[...]

    return pl.pallas_call(
        body, grid=(GN,),
        in_specs=[_nblk(D), _wspec(D, D), _wspec(1, D),
                  _wspec(D, OUT), _wspec(1, OUT)],
        out_specs=_nblk(OUT), out_shape=_f32(NP, OUT),
    )(hp, w1, b1, w2, b2)


def kernel(h, pos, edge_index, emb_w, emb_b, msg_w1, msg_b1, msg_w2, msg_b2,
           coord_w1, coord_b1, coord_w2, coord_b2, node_w1, node_b1,
           node_w2, node_b2, out_w1, out_b1, out_w2, out_b2):
    pad_idx = jnp.full((EP - E,), N, jnp.int32)
    srcp = jnp.concatenate([edge_index[0], pad_idx]).reshape(EP // K, K)
    dstp = jnp.concatenate([edge_index[1], pad_idx]).reshape(EP // K, K)
    hp = jnp.pad(h, ((0, NP - N), (0, 0)))
    p8 = jnp.pad(pos, ((0, NP - N), (0, P8 - 3)))
    zd = jnp.zeros((K, D), jnp.float32)
    zp = jnp.zeros((K, P8), jnp.float32)

    he = _tc_embed(hp, emb_w, emb_b.reshape(1, D))
    for l in range(L):
        w1 = msg_w1[l]
        wd = w1[2 * D].reshape(1, D)
        b1 = msg_b1[l].reshape(1, D)
        b2 = msg_b2[l].reshape(1, D)
        a_t, b_t, mself, posc8 = _tc_prep(
            he, p8, w1[:D], w1[D:2 * D], wd, b1, msg_w2[l], b2,
            coord_w1[l], coord_b1[l].reshape(1, D),
            coord_w2[l], coord_b2[l].reshape(1, 1))
        s, ps, pd = _sc_gather(srcp, dstp, a_t, b_t, posc8, p8)
        m2, cd = _tc_edge(s, ps, pd, wd, b1, msg_w2[l], b2)
        aggp, dposp = _sc_scatter(m2, cd, dstp, zd, zp)
        he, p8 = _tc_node(
            he, aggp, mself, dposp, p8,
            node_w1[l][:D], node_w1[l][D:], node_b1[l].reshape(1, D),
            node_w2[l], node_b2[l].reshape(1, D))

    featp = _tc_head(he, out_w1, out_b1.reshape(1, D),
                     out_w2, out_b2.reshape(1, OUT))
    return featp[:N], p8[:N, :3]


# speedup vs baseline: 2.0363x; 2.0363x over previous
import functools

import jax
import jax.numpy as jnp
from jax import lax
from jax.experimental import pallas as pl
from jax.experimental.pallas import tpu as pltpu
from jax.experimental.pallas import tpu_sc as plsc

N = 10000
D = 128
E = 160000
L = 4
OUT = 128

NP = 10240
EP = 163840
NC = 2
NS = 16
NW = NC * NS
PER_TILE = EP // NW
K = 128
CHUNKS = PER_TILE // K
CH0 = 54
CH1 = 80 - CH0
CHMAX = max(CH0, CH1)
NCHROW = EP // K
GDEP = 3
RPT = NP // NS
P8 = 8

_mesh = plsc.VectorSubcoreMesh(core_axis_name="c", subcore_axis_name="s")


def _f32(*shape):
    return jax.ShapeDtypeStruct(shape, jnp.float32)


def _silu(x):
    return x * jax.nn.sigmoid(x)


@functools.partial(
    pl.kernel,
    out_type=(_f32(EP, D), _f32(EP, P8), _f32(EP, P8)),
    mesh=_mesh,
    compiler_params=pltpu.CompilerParams(use_tc_tiling_on_sc=False),
    scratch_types=(
        pltpu.VMEM((CHMAX, K), jnp.int32),
        pltpu.VMEM((CHMAX, K), jnp.int32),
        pltpu.VMEM((GDEP, K, D), jnp.float32),
        pltpu.VMEM((GDEP, K, D), jnp.float32),
        pltpu.VMEM((GDEP, K, P8), jnp.float32),
        pltpu.VMEM((GDEP, K, P8), jnp.float32),
        pltpu.SemaphoreType.DMA((GDEP,)),
        pltpu.SemaphoreType.DMA((GDEP,)),
        pltpu.SemaphoreType.DMA,
    ),
)
def _sc_gather(srcp2, dstp2, a_t, b_t, ps_t, pd_t, s_out, ps_out, pd_out,
               sidxs, didxs, abuf, bbuf, psbuf, pdbuf, gs, os_, isem):
    cid = lax.axis_index("c")
    sid = lax.axis_index("s")
    ch_mine = jnp.where(cid == 0, CH0, CH1)
    crow = sid * ch_mine + cid * (NS * CH0)
    base = crow * K
    crow_ld = jnp.minimum(crow, NCHROW - CHMAX)

    ci1 = pltpu.async_copy(srcp2.at[pl.ds(crow_ld, CHMAX)], sidxs, isem)
    ci2 = pltpu.async_copy(dstp2.at[pl.ds(crow_ld, CHMAX)], didxs, isem)
    ci1.wait()
    ci2.wait()

    def fire(ci, b):
        pltpu.async_copy(a_t.at[didxs.at[ci]], abuf.at[b], gs.at[b])
        pltpu.async_copy(b_t.at[sidxs.at[ci]], bbuf.at[b], gs.at[b])
        pltpu.async_copy(ps_t.at[sidxs.at[ci]], psbuf.at[b], gs.at[b])
        pltpu.async_copy(pd_t.at[didxs.at[ci]], pdbuf.at[b], gs.at[b])

    def wait_gathers(ci, b):
        pltpu.make_async_copy(a_t.at[didxs.at[ci]], abuf.at[b], gs.at[b]).wait()
        pltpu.make_async_copy(b_t.at[sidxs.at[ci]], bbuf.at[b], gs.at[b]).wait()
        pltpu.make_async_copy(ps_t.at[sidxs.at[ci]], psbuf.at[b], gs.at[b]).wait()
        pltpu.make_async_copy(pd_t.at[didxs.at[ci]], pdbuf.at[b], gs.at[b]).wait()

    for b in range(GDEP):
        @pl.when(b < ch_mine)
        def _():
            fire(b, b)

    @pl.loop(0, (CHMAX + GDEP - 1) // GDEP)
    def _round(r):
        for b in range(GDEP):
            ci = r * GDEP + b

            @pl.when(ci < ch_mine)
            def _():
                wait_gathers(ci, b)

                @pl.loop(0, K, unroll=4)
                def _row(e):
                    for c in range(D // 16):
                        sl = pl.ds(c * 16, 16)
                        abuf[b, e, sl] = abuf[b, e, sl] + bbuf[b, e, sl]

                off = base + ci * K
                o1 = pltpu.async_copy(abuf.at[b], s_out.at[pl.ds(off, K)], os_.at[b])
                o2 = pltpu.async_copy(psbuf.at[b], ps_out.at[pl.ds(off, K)], os_.at[b])
                o3 = pltpu.async_copy(pdbuf.at[b], pd_out.at[pl.ds(off, K)], os_.at[b])

                @pl.when(ci + GDEP < ch_mine)
                def _():
                    o1.wait()
                    o2.wait()
                    o3.wait()
                    fire(ci + GDEP, b)

    for b in range(GDEP):
        @pl.when(b < ch_mine)
        def _():
            pltpu.make_async_copy(abuf.at[b], s_out.at[pl.ds(base, K)], os_.at[b]).wait()
            pltpu.make_async_copy(psbuf.at[b], ps_out.at[pl.ds(base, K)], os_.at[b]).wait()
            pltpu.make_async_copy(pdbuf.at[b], pd_out.at[pl.ds(base, K)], os_.at[b]).wait()


@functools.partial(
    pl.kernel,
    out_type=(_f32(NC, NP, D), _f32(NC, NP, P8)),
    mesh=_mesh,
    compiler_params=pltpu.CompilerParams(use_tc_tiling_on_sc=False),
    scratch_types=(
        pltpu.VMEM((CHUNKS, K), jnp.int32),
        pltpu.VMEM((K, D), jnp.float32),
        pltpu.VMEM((K, D), jnp.float32),
        pltpu.VMEM((K, P8), jnp.float32),
        pltpu.VMEM((K, P8), jnp.float32),
        pltpu.VMEM_SHARED((NP, D), jnp.float32),
        pltpu.VMEM_SHARED((NP, P8), jnp.float32),
        pltpu.SemaphoreType.DMA,
        pltpu.SemaphoreType.DMA,
        pltpu.SemaphoreType.DMA,
    ),
)
def _sc_scatter(m2, cd, dstp2, zd, zp, agg_out, dpos_out,
                didxs, mb0, mb1, cb0, cb1, agg_sh, dpos_sh, ls0, ls1, isem):
    cid = lax.axis_index("c")
    sid = lax.axis_index("s")
    wid = sid * NC + cid
    base = wid * PER_TILE
    crow = wid * CHUNKS
    row0 = sid * RPT

    ci = pltpu.async_copy(dstp2.at[pl.ds(crow, CHUNKS)], didxs, isem)

    pltpu.sync_copy(zd, mb0)
    pltpu.sync_copy(zp, cb0)
    for j in range(RPT // K):
        pltpu.sync_copy(mb0, agg_sh.at[pl.ds(row0 + j * K, K)])
        pltpu.sync_copy(cb0, dpos_sh.at[pl.ds(row0 + j * K, K)])
    ci.wait()
    plsc.subcore_barrier()

    bufs = ((mb0, cb0, ls0), (mb1, cb1, ls1))

    def fire(ci_, b):
        mb, cb, ls = bufs[b]
        off = base + ci_ * K
        return (pltpu.async_copy(m2.at[pl.ds(off, K)], mb, ls),
                pltpu.async_copy(cd.at[pl.ds(off, K)], cb, ls))

    def consume(ci_, b, lds):
        mb, cb, _ = bufs[b]
        for l_ in lds:
            l_.wait()
        pltpu.sync_copy(mb, agg_sh.at[didxs.at[ci_]], add=True)
        pltpu.sync_copy(cb, dpos_sh.at[didxs.at[ci_]], add=True)

    @pl.loop(0, CHUNKS // 2)
    def _pair(j):
        c0 = 2 * j
        c1 = c0 + 1
        l0 = fire(c0, 0)
        l1 = fire(c1, 1)
        consume(c0, 0, l0)
        consume(c1, 1, l1)

    plsc.subcore_barrier()
    for j in range(RPT // K):
        r = row0 + j * K
        pltpu.sync_copy(agg_sh.at[pl.ds(r, K)], mb0)
        pltpu.sync_copy(mb0, agg_out.at[cid, pl.ds(r, K)])
        pltpu.sync_copy(dpos_sh.at[pl.ds(r, K)], cb0)
        pltpu.sync_copy(cb0, dpos_out.at[cid, pl.ds(r, K)])


BN = 1024
GN = NP // BN
BE = 2048
GE = EP // BE


def _wspec(r, c):
    return pl.BlockSpec((r, c), lambda i: (0, 0))


def _nblk(c):
    return pl.BlockSpec((BN, c), lambda i: (i, 0))


def _eblk(c):
    return pl.BlockSpec((BE, c), lambda i: (i, 0))


def _dot(x, w):
    return jnp.dot(x, w, preferred_element_type=jnp.float32)


def _bf(x):
    return x.astype(jnp.bfloat16).astype(jnp.float32)


def _tc_embed(hp, w, b):
    def body(h_ref, w_ref, b_ref, o_ref):
        i = pl.program_id(0)
        x = _dot(h_ref[...], w_ref[...]) + b_ref[...]
        rid = i * BN + lax.broadcasted_iota(jnp.int32, (BN, 1), 0)
        o_ref[...] = jnp.where(rid < N, x, 0.0)

    return pl.pallas_call(
        body, grid=(GN,),
        in_specs=[_nblk(D), _wspec(D, D), _wspec(1, D)],
        out_specs=_nblk(D), out_shape=_f32(NP, D),
    )(hp, w, b)


def _tc_prep(hp, p8, w1a, w1b, wd, b1, w2, b2, cw1, cb1, cw2, cb2):
    def body(h_ref, p8_ref, w1a_r, w1b_r, wd_r, b1_r, w2_r, b2_r,
             cw1_r, cb1_r, cw2_r, cb2_r, a_o, b_o, ms_o, pc_o):
        x = h_ref[...]
        a = _dot(x, w1a_r[...])
        b = _dot(x, w1b_r[...])
        a_o[...] = a
        b_o[...] = b
        pre = _silu(a + b + _bf(1e-10 * jnp.ones((1, 1), jnp.float32)) * _bf(wd_r[...]) + b1_r[...])
        ms_o[...] = _silu(_dot(pre, w2_r[...]) + b2_r[...])
        cvec = _silu(_dot(x, cw1_r[...]) + cb1_r[...])
        cval = _dot(cvec, cw2_r[...]) + cb2_r[...]
        lane = lax.broadcasted_iota(jnp.int32, (BN, P8), 1)
        pc_o[...] = jnp.where(lane == 3, cval, p8_ref[...])

    return pl.pallas_call(
        body, grid=(GN,),
        in_specs=[_nblk(D), _nblk(P8), _wspec(D, D), _wspec(D, D),
                  _wspec(1, D), _wspec(1, D), _wspec(D, D), _wspec(1, D),
                  _wspec(D, D), _wspec(1, D), _wspec(D, 1), _wspec(1, 1)],
        out_specs=[_nblk(D), _nblk(D), _nblk(D), _nblk(P8)],
        out_shape=[_f32(NP, D), _f32(NP, D), _f32(NP, D), _f32(NP, P8)],
    )(hp, p8, w1a, w1b, wd, b1, w2, b2, cw1, cb1, cw2, cb2)


def _tc_edge(s, ps, pd, wd, b1, w2, b2):
    def body(s_ref, ps_ref, pd_ref, wd_r, b1_r, w2_r, b2_r, m2_o, cd_o):
        ps_v = ps_ref[...]
        pd_v = pd_ref[...]
        lane = lax.broadcasted_iota(jnp.int32, (BE, P8), 1)
        rel = jnp.where(lane < 3, ps_v - pd_v, 0.0)
        d2 = jnp.sum(rel * rel, axis=1, keepdims=True) + 1e-20
        dist = jnp.sqrt(d2)
        m1 = _silu(s_ref[...] + _bf(dist) * _bf(wd_r[...]) + b1_r[...])
        m2_o[...] = _silu(_dot(m1, w2_r[...]) + b2_r[...])
        cw = jnp.sum(jnp.where(lane == 3, ps_v, 0.0), axis=1, keepdims=True)
        cd_o[...] = cw * rel / (dist + 1e-8)

    return pl.pallas_call(
        body, grid=(GE,),
        in_specs=[_eblk(D), _eblk(P8), _eblk(P8),
                  _wspec(1, D), _wspec(1, D), _wspec(D, D), _wspec(1, D)],
        out_specs=[_eblk(D), _eblk(P8)],
        out_shape=[_f32(EP, D), _f32(EP, P8)],
    )(s, ps, pd, wd, b1, w2, b2)


def _tc_node(hp, aggp, mself, dposp, p8, nw1a, nw1b, nb1, nw2, nb2):
    def body(h_ref, agg_ref, ms_ref, dp_ref, p8_ref,
             nw1a_r, nw1b_r, nb1_r, nw2_r, nb2_r, h_o, p_o):
        i = pl.program_id(0)
        h = h_ref[...]
        agg = agg_ref[0] + agg_ref[1] + ms_ref[...]
        t = _silu(_dot(h, nw1a_r[...]) + _dot(agg, nw1b_r[...]) + nb1_r[...])
        upd = _dot(t, nw2_r[...]) + nb2_r[...]
        rid = i * BN + lax.broadcasted_iota(jnp.int32, (BN, 1), 0)
        mask = rid < N
        h_o[...] = h + jnp.where(mask, upd, 0.0)
        dp = dp_ref[0] + dp_ref[1]
        p_o[...] = p8_ref[...] + jnp.where(mask, dp, 0.0)

    return pl.pallas_call(
        body, grid=(GN,),
        in_specs=[_nblk(D),
                  pl.BlockSpec((NC, BN, D), lambda i: (0, i, 0)),
                  _nblk(D),
                  pl.BlockSpec((NC, BN, P8), lambda i: (0, i, 0)),
                  _nblk(P8),
                  _wspec(D, D), _wspec(D, D), _wspec(1, D),
                  _wspec(D, D), _wspec(1, D)],
        out_specs=[_nblk(D), _nblk(P8)],
        out_shape=[_f32(NP, D), _f32(NP, P8)],
    )(hp, aggp, mself, dposp, p8, nw1a, nw1b, nb1, nw2, nb2)


def _tc_head(hp, w1, b1, w2, b2):
    def body(h_ref, w1_r, b1_r, w2_r, b2_r, o_ref):
        t = _silu(_dot(h_ref[...], w1_r[...]) + b1_r[...])
        o_ref[...] = _dot(t, w2_r[...]) + b2_r[...]

    return pl.pallas_call(
        body, grid=(GN,),
        in_specs=[_nblk(D), _wspec(D, D), _wspec(1, D),
                  _wspec(D, OUT), _wspec(1, OUT)],
        out_specs=_nblk(OUT), out_shape=_f32(NP, OUT),
    )(hp, w1, b1, w2, b2)


def kernel(h, pos, edge_index, emb_w, emb_b, msg_w1, msg_b1, msg_w2, msg_b2,
           coord_w1, coord_b1, coord_w2, coord_b2, node_w1, node_b1,
           node_w2, node_b2, out_w1, out_b1, out_w2, out_b2):
    pad_idx = jnp.full((EP - E,), N, jnp.int32)
    srcp = jnp.concatenate([edge_index[0], pad_idx]).reshape(EP // K, K)
    dstp = jnp.concatenate([edge_index[1], pad_idx]).reshape(EP // K, K)
    hp = jnp.pad(h, ((0, NP - N), (0, 0)))
    p8 = jnp.pad(pos, ((0, NP - N), (0, P8 - 3)))
    zd = jnp.zeros((K, D), jnp.float32)
    zp = jnp.zeros((K, P8), jnp.float32)

    he = _tc_embed(hp, emb_w, emb_b.reshape(1, D))
    for l in range(L):
        w1 = msg_w1[l]
        wd = w1[2 * D].reshape(1, D)
        b1 = msg_b1[l].reshape(1, D)
        b2 = msg_b2[l].reshape(1, D)
        a_t, b_t, mself, posc8 = _tc_prep(
            he, p8, w1[:D], w1[D:2 * D], wd, b1, msg_w2[l], b2,
            coord_w1[l], coord_b1[l].reshape(1, D),
            coord_w2[l], coord_b2[l].reshape(1, 1))
        s, ps, pd = _sc_gather(srcp, dstp, a_t, b_t, posc8, p8)
        m2, cd = _tc_edge(s, ps, pd, wd, b1, msg_w2[l], b2)
        aggp, dposp = _sc_scatter(m2, cd, dstp, zd, zp)
        he, p8 = _tc_node(
            he, aggp, mself, dposp, p8,
            node_w1[l][:D], node_w1[l][D:], node_b1[l].reshape(1, D),
            node_w2[l], node_b2[l].reshape(1, D))

    featp = _tc_head(he, out_w1, out_b1.reshape(1, D),
                     out_w2, out_b2.reshape(1, OUT))
    return featp[:N], p8[:N, :3]
